# Initial kernel scaffold; baseline (speedup 1.0000x reference)
import jax
import jax.numpy as jnp
from jax.experimental import pallas as pl


def kernel(X, card, rank, suit):
    raise NotImplementedError("write your pallas kernel here")


# speedup vs baseline: 4.5080x; 4.5080x over previous
import functools

import jax
import jax.numpy as jnp
from jax import lax
from jax.experimental import pallas as pl
from jax.experimental.pallas import tpu as pltpu
from jax.experimental.pallas import tpu_sc as plsc

BATCH = 16384
NCARDS = 7
DIM = 128
NCHUNK = DIM // 16

NC = 2
NS = 16
NW = NC * NS
BPW = BATCH // NW
GROUPS = BPW // 16
DUNROLL = 4


def _sc_body(x_hbm, card_hbm, rank_hbm, suit_hbm, out_hbm,
             card_v, rank_v, suit_v, t_v, x_v, out_v):
    wid = lax.axis_index("s") * NC + lax.axis_index("c")
    pltpu.sync_copy(card_hbm, card_v)
    pltpu.sync_copy(rank_hbm, rank_v)
    pltpu.sync_copy(suit_hbm, suit_v)
    pltpu.sync_copy(x_hbm.at[pl.ds(wid * (BPW * NCARDS), BPW * NCARDS)], x_v)

    def build(i, carry):
        q = i // 4
        m = i - q * 4
        for j in range(NCHUNK):
            js = pl.ds(j * 16, 16)
            t_v[i, js] = card_v[i, js] + rank_v[q, js] + suit_v[m, js]
        return carry

    lax.fori_loop(1, 52, build, 0)
    zero = jnp.zeros((16,), jnp.float32)
    for j in range(NCHUNK):
        t_v[0, pl.ds(j * 16, 16)] = zero

    lane = lax.iota(jnp.int32, 16)
    lane7 = lane * NCARDS

    def group(g, carry):
        gbase = g * (16 * NCARDS)
        idxs = [plsc.load_gather(x_v, [lane7 + (gbase + c)])
                for c in range(NCARDS)]
        rows = lane + g * 16

        def dloop(du, carry2):
            for u in range(DUNROLL):
                d = du * DUNROLL + u
                dvec = jnp.full((16,), d, jnp.int32)
                acc = plsc.load_gather(t_v, [idxs[0], dvec])
                for c in range(1, NCARDS):
                    acc = acc + plsc.load_gather(t_v, [idxs[c], dvec])
                plsc.store_scatter(out_v, [rows, dvec], acc)
            return carry2

        lax.fori_loop(0, DIM // DUNROLL, dloop, 0)
        return carry

    lax.fori_loop(0, GROUPS, group, 0)
    pltpu.sync_copy(out_v, out_hbm.at[pl.ds(wid * BPW, BPW)])


@jax.jit
def kernel(X, card, rank, suit):
    x_flat = X.reshape(-1).astype(jnp.int32)
    f = pl.kernel(
        _sc_body,
        out_type=jax.ShapeDtypeStruct((BATCH, DIM), jnp.float32),
        mesh=plsc.VectorSubcoreMesh(core_axis_name="c", subcore_axis_name="s"),
        compiler_params=pltpu.CompilerParams(needs_layout_passes=False),
        scratch_types=[
            pltpu.VMEM((52, DIM), jnp.float32),
            pltpu.VMEM((13, DIM), jnp.float32),
            pltpu.VMEM((4, DIM), jnp.float32),
            pltpu.VMEM((52, DIM), jnp.float32),
            pltpu.VMEM((BPW * NCARDS,), jnp.int32),
            pltpu.VMEM((BPW, DIM), jnp.float32),
        ],
    )
    return f(x_flat, card.astype(jnp.float32), rank.astype(jnp.float32),
             suit.astype(jnp.float32))


# speedup vs baseline: 5.1287x; 1.1377x over previous
import jax
import jax.numpy as jnp
from jax import lax
from jax.experimental import pallas as pl
from jax.experimental.pallas import tpu as pltpu
from jax.experimental.pallas import tpu_sc as plsc

BATCH = 16384
NCARDS = 7
DIM = 128
NCHUNK = DIM // 16

NC = 2
NS = 16
NW = NC * NS
BPW = BATCH // NW
GROUPS = BPW // 16


def _sc_body(x_hbm, card_hbm, rank_hbm, suit_hbm, out_hbm,
             card_v, rank_v, suit_v, t_v, x_v, out_v):
    wid = lax.axis_index("s") * NC + lax.axis_index("c")
    pltpu.sync_copy(card_hbm, card_v)
    pltpu.sync_copy(rank_hbm, rank_v)
    pltpu.sync_copy(suit_hbm, suit_v)
    pltpu.sync_copy(x_hbm.at[pl.ds(wid * (BPW * NCARDS), BPW * NCARDS)], x_v)

    @plsc.parallel_loop(1, 52)
    def _build(i):
        q = i // 4
        m = i - q * 4
        for j in range(NCHUNK):
            t_v[pl.ds(i * DIM + j * 16, 16)] = (
                card_v[pl.ds(i * DIM + j * 16, 16)]
                + rank_v[pl.ds(q * DIM + j * 16, 16)]
                + suit_v[pl.ds(m * DIM + j * 16, 16)])

    zero = jnp.zeros((16,), jnp.float32)
    for j in range(NCHUNK):
        t_v[pl.ds(j * 16, 16)] = zero

    lane = lax.iota(jnp.int32, 16)
    lane7 = lane * NCARDS
    lane128 = lane * DIM

    @plsc.parallel_loop(0, GROUPS)
    def _group(g):
        gbase = g * (16 * NCARDS)
        fidx = [plsc.load_gather(x_v, [lane7 + (gbase + c)]) * DIM
                for c in range(NCARDS)]
        outbase = lane128 + g * (16 * DIM)

        @plsc.parallel_loop(0, DIM, unroll=8)
        def _dloop(d):
            dvec = jnp.full((16,), d, jnp.int32)
            g0 = plsc.load_gather(t_v, [fidx[0] + dvec])
            g1 = plsc.load_gather(t_v, [fidx[1] + dvec])
            g2 = plsc.load_gather(t_v, [fidx[2] + dvec])
            g3 = plsc.load_gather(t_v, [fidx[3] + dvec])
            g4 = plsc.load_gather(t_v, [fidx[4] + dvec])
            g5 = plsc.load_gather(t_v, [fidx[5] + dvec])
            g6 = plsc.load_gather(t_v, [fidx[6] + dvec])
            acc = ((g0 + g1) + (g2 + g3)) + ((g4 + g5) + g6)
            plsc.store_scatter(out_v, [outbase + dvec], acc)

    pltpu.sync_copy(out_v, out_hbm.at[pl.ds(wid * (BPW * DIM), BPW * DIM)])


@jax.jit
def kernel(X, card, rank, suit):
    x_flat = X.reshape(-1).astype(jnp.int32)
    f = pl.kernel(
        _sc_body,
        out_type=jax.ShapeDtypeStruct((BATCH * DIM,), jnp.float32),
        mesh=plsc.VectorSubcoreMesh(core_axis_name="c", subcore_axis_name="s"),
        compiler_params=pltpu.CompilerParams(needs_layout_passes=False),
        scratch_types=[
            pltpu.VMEM((52 * DIM,), jnp.float32),
            pltpu.VMEM((13 * DIM,), jnp.float32),
            pltpu.VMEM((4 * DIM,), jnp.float32),
            pltpu.VMEM((52 * DIM,), jnp.float32),
            pltpu.VMEM((BPW * NCARDS,), jnp.int32),
            pltpu.VMEM((BPW * DIM,), jnp.float32),
        ],
    )
    out = f(x_flat, card.astype(jnp.float32).reshape(-1),
            rank.astype(jnp.float32).reshape(-1),
            suit.astype(jnp.float32).reshape(-1))
    return out.reshape(BATCH, DIM)


# speedup vs baseline: 27.5490x; 5.3716x over previous
import jax
import jax.numpy as jnp
from jax import lax
from jax.experimental import pallas as pl
from jax.experimental.pallas import tpu as pltpu
from jax.experimental.pallas import tpu_sc as plsc

BATCH = 16384
NCARDS = 7
DIM = 128
NCHUNK = DIM // 16

NC = 2
NS = 16
NW = NC * NS
BPW = BATCH // NW
XPW = BPW * NCARDS


def _sc_body(x_hbm, card_hbm, rank_hbm, suit_hbm, out_hbm,
             card_v, rank_v, suit_v, t_v, x_v, out_v):
    wid = lax.axis_index("s") * NC + lax.axis_index("c")
    pltpu.sync_copy(card_hbm, card_v)
    pltpu.sync_copy(rank_hbm, rank_v)
    pltpu.sync_copy(suit_hbm, suit_v)
    pltpu.sync_copy(x_hbm.at[pl.ds(wid * XPW, XPW)], x_v.at[pl.ds(0, XPW)])

    @plsc.parallel_loop(1, 52)
    def _build(i):
        q = i // 4
        m = i - q * 4
        for j in range(NCHUNK):
            t_v[pl.ds(i * DIM + j * 16, 16)] = (
                card_v[pl.ds(i * DIM + j * 16, 16)]
                + rank_v[pl.ds(q * DIM + j * 16, 16)]
                + suit_v[pl.ds(m * DIM + j * 16, 16)])

    zero = jnp.zeros((16,), jnp.float32)
    for j in range(NCHUNK):
        t_v[pl.ds(j * 16, 16)] = zero

    @plsc.parallel_loop(0, BPW, unroll=2)
    def _row(b):
        xrow = x_v[pl.ds(b * NCARDS, 16)]
        base = [xrow[c] * DIM for c in range(NCARDS)]
        obase = b * DIM
        for j in range(NCHUNK):
            js = j * 16
            t0 = t_v[pl.ds(base[0] + js, 16)]
            t1 = t_v[pl.ds(base[1] + js, 16)]
            t2 = t_v[pl.ds(base[2] + js, 16)]
            t3 = t_v[pl.ds(base[3] + js, 16)]
            t4 = t_v[pl.ds(base[4] + js, 16)]
            t5 = t_v[pl.ds(base[5] + js, 16)]
            t6 = t_v[pl.ds(base[6] + js, 16)]
            out_v[pl.ds(obase + js, 16)] = ((t0 + t1) + (t2 + t3)) + ((t4 + t5) + t6)

    pltpu.sync_copy(out_v, out_hbm.at[pl.ds(wid * (BPW * DIM), BPW * DIM)])


@jax.jit
def kernel(X, card, rank, suit):
    x_flat = X.reshape(-1).astype(jnp.int32)
    f = pl.kernel(
        _sc_body,
        out_type=jax.ShapeDtypeStruct((BATCH * DIM,), jnp.float32),
        mesh=plsc.VectorSubcoreMesh(core_axis_name="c", subcore_axis_name="s"),
        compiler_params=pltpu.CompilerParams(needs_layout_passes=False),
        scratch_types=[
            pltpu.VMEM((52 * DIM,), jnp.float32),
            pltpu.VMEM((13 * DIM,), jnp.float32),
            pltpu.VMEM((4 * DIM,), jnp.float32),
            pltpu.VMEM((52 * DIM,), jnp.float32),
            pltpu.VMEM((XPW + 16,), jnp.int32),
            pltpu.VMEM((BPW * DIM,), jnp.float32),
        ],
    )
    out = f(x_flat, card.astype(jnp.float32).reshape(-1),
            rank.astype(jnp.float32).reshape(-1),
            suit.astype(jnp.float32).reshape(-1))
    return out.reshape(BATCH, DIM)
